# baseline (speedup 1.0000x reference)
import jax
import jax.numpy as jnp
from jax import lax
from jax.experimental import pallas as pl
from jax.experimental.pallas import tpu as pltpu
from jax.experimental.pallas import tpu_sc as plsc

N = 10000
T = 4
E = 160000
NSH = 16
ESH = E // NSH
NCHUNK = 79
EPAD = NCHUNK * 128
PADDST = N
ACCN = 10240
ZCH = ACCN // 16
NSL = 632
NBUF = 640
LASTSTART = N - NSL
RING = 4


def _fisr(x):
    i = lax.bitcast_convert_type(x, jnp.int32)
    i = jnp.int32(0x5F3759DF) - lax.shift_right_arithmetic(i, 1)
    y = lax.bitcast_convert_type(i, jnp.float32)
    for _ in range(3):
        y = y * (1.5 - 0.5 * x * y * y)
    return y


def _ring_scatter(n, make_copy):
    for j in range(RING):
        make_copy(j).start(add=True)

    def body(j, _):
        make_copy(j).start(add=True)
        make_copy(j - RING).wait()
        return 0
    lax.fori_loop(RING, n, body, 0)
    for j in range(n - RING, n):
        make_copy(j).wait()


def _sc_gcn_body(src_hbm, dst_hbm, x0_hbm, x1_hbm, wv_hbm,
                 out0_hbm, out1_hbm,
                 idx2d, src2d, val0, val1, g0v, g1v,
                 x0s, x1s, dinvs, gt0, gt1, a0s, a1s, outs, zbuf, wv,
                 sem0, sem1,
                 deg_sh, acc0_sh, acc1_sh, g0_sh, g1_sh, *, pair):
    c = lax.axis_index("c")
    s = lax.axis_index("s")
    t = 2 * pair + c
    start = lax.min(s * NSL, LASTSTART)

    ones16 = jnp.full((16,), 1.0, dtype=jnp.float32)
    zero16 = jnp.zeros((16,), dtype=jnp.float32)

    def zfill(m, _):
        zbuf[pl.ds(m * 16, 16)] = zero16
        return 0
    lax.fori_loop(0, ZCH // 16, zfill, 0)
    pltpu.sync_copy(zbuf, deg_sh.at[pl.ds(s * ZCH, ZCH)])
    pltpu.sync_copy(zbuf, acc0_sh.at[pl.ds(s * ZCH, ZCH)])
    pltpu.sync_copy(zbuf, acc1_sh.at[pl.ds(s * ZCH, ZCH)])

    pltpu.sync_copy(wv_hbm, wv)
    chunk = t * NSH + s
    pltpu.sync_copy(dst_hbm.at[chunk], idx2d)
    pltpu.sync_copy(src_hbm.at[chunk], src2d)
    nbase = t * N + start
    pltpu.sync_copy(x0_hbm.at[pl.ds(nbase, NSL)], x0s.at[pl.ds(0, NSL)])
    pltpu.sync_copy(x1_hbm.at[pl.ds(nbase, NSL)], x1s.at[pl.ds(0, NSL)])

    def onesfill(j, _):
        for k in range(8):
            val0[j, pl.ds(k * 16, 16)] = ones16
        return 0
    lax.fori_loop(0, NCHUNK, onesfill, 0)

    plsc.subcore_barrier()

    def deg_copy(j):
        return pltpu.make_async_copy(val0.at[j], deg_sh.at[idx2d.at[j]], sem0)
    _ring_scatter(NCHUNK, deg_copy)

    plsc.subcore_barrier()

    pltpu.sync_copy(deg_sh.at[pl.ds(start, NSL)], dinvs.at[pl.ds(0, NSL)])
    wvec = wv[pl.ds(0, 16)]
    w00 = wvec[0]
    w01 = wvec[1]
    w10 = wvec[2]
    w11 = wvec[3]

    def gbody(m, _):
        dv = dinvs[pl.ds(m * 16, 16)]
        di = _fisr(dv + 1.0)
        dinvs[pl.ds(m * 16, 16)] = di
        xa = x0s[pl.ds(m * 16, 16)]
        xb = x1s[pl.ds(m * 16, 16)]
        gt0[pl.ds(m * 16, 16)] = (xa * w00 + xb * w01) * di
        gt1[pl.ds(m * 16, 16)] = (xa * w10 + xb * w11) * di
        return 0
    lax.fori_loop(0, NBUF // 16, gbody, 0)
    pltpu.sync_copy(gt0.at[pl.ds(0, NSL)], g0_sh.at[pl.ds(start, NSL)])
    pltpu.sync_copy(gt1.at[pl.ds(0, NSL)], g1_sh.at[pl.ds(start, NSL)])

    plsc.subcore_barrier()

    pltpu.sync_copy(g0_sh, g0v)
    pltpu.sync_copy(g1_sh, g1v)

    def gathbody(j, _):
        for k in range(8):
            sv = src2d[j, pl.ds(k * 16, 16)]
            val0[j, pl.ds(k * 16, 16)] = plsc.load_gather(g0v, [sv])
            val1[j, pl.ds(k * 16, 16)] = plsc.load_gather(g1v, [sv])
        return 0
    lax.fori_loop(0, NCHUNK, gathbody, 0)

    def msg_copy0(j):
        return pltpu.make_async_copy(val0.at[j], acc0_sh.at[idx2d.at[j]], sem0)

    def msg_copy1(j):
        return pltpu.make_async_copy(val1.at[j], acc1_sh.at[idx2d.at[j]], sem1)
    _ring_scatter(NCHUNK, msg_copy0)
    _ring_scatter(NCHUNK, msg_copy1)

    plsc.subcore_barrier()

    pltpu.sync_copy(acc0_sh.at[pl.ds(start, NSL)], a0s.at[pl.ds(0, NSL)])
    pltpu.sync_copy(acc1_sh.at[pl.ds(start, NSL)], a1s.at[pl.ds(0, NSL)])
    bvec = wv[pl.ds(0, 16)]
    b0 = bvec[4]
    b1 = bvec[5]
    obase = (t - 2 * pair) * N + start

    def obody(m, _):
        di = dinvs[pl.ds(m * 16, 16)]
        outs[pl.ds(m * 16, 16)] = (
            di * (a0s[pl.ds(m * 16, 16)] + gt0[pl.ds(m * 16, 16)]) + b0)
        return 0
    lax.fori_loop(0, NBUF // 16, obody, 0)
    pltpu.sync_copy(outs.at[pl.ds(0, NSL)], out0_hbm.at[pl.ds(obase, NSL)])

    def obody1(m, _):
        di = dinvs[pl.ds(m * 16, 16)]
        outs[pl.ds(m * 16, 16)] = (
            di * (a1s[pl.ds(m * 16, 16)] + gt1[pl.ds(m * 16, 16)]) + b1)
        return 0
    lax.fori_loop(0, NBUF // 16, obody1, 0)
    pltpu.sync_copy(outs.at[pl.ds(0, NSL)], out1_hbm.at[pl.ds(obase, NSL)])


def _sc_gcn(src_p, dst_p, x0, x1, wv, pair):
    mesh = plsc.VectorSubcoreMesh(core_axis_name="c", subcore_axis_name="s",
                                  num_cores=2, num_subcores=16)
    f32 = jnp.float32

    def body(*refs):
        _sc_gcn_body(*refs, pair=pair)

    fn = pl.kernel(
        body,
        out_type=(jax.ShapeDtypeStruct((2 * N,), f32),
                  jax.ShapeDtypeStruct((2 * N,), f32)),
        mesh=mesh,
        compiler_params=pltpu.CompilerParams(needs_layout_passes=False),
        scratch_types=[
            pltpu.VMEM((NCHUNK, 128), jnp.int32),
            pltpu.VMEM((NCHUNK, 128), jnp.int32),
            pltpu.VMEM((NCHUNK, 128), f32),
            pltpu.VMEM((NCHUNK, 128), f32),
            pltpu.VMEM((N,), f32),
            pltpu.VMEM((N,), f32),
            pltpu.VMEM((NBUF,), f32),
            pltpu.VMEM((NBUF,), f32),
            pltpu.VMEM((NBUF,), f32),
            pltpu.VMEM((NBUF,), f32),
            pltpu.VMEM((NBUF,), f32),
            pltpu.VMEM((NBUF,), f32),
            pltpu.VMEM((NBUF,), f32),
            pltpu.VMEM((NBUF,), f32),
            pltpu.VMEM((ZCH,), f32),
            pltpu.VMEM((16,), f32),
            pltpu.SemaphoreType.DMA,
            pltpu.SemaphoreType.DMA,
            pltpu.VMEM_SHARED((ACCN,), f32),
            pltpu.VMEM_SHARED((ACCN,), f32),
            pltpu.VMEM_SHARED((ACCN,), f32),
            pltpu.VMEM_SHARED((N,), f32),
            pltpu.VMEM_SHARED((N,), f32),
        ],
    )
    return fn(src_p, dst_p, x0, x1, wv)


def _tc_rnn_body(c0a_ref, c1a_ref, c0b_ref, c1b_ref,
                 wa_ref, wb_ref, ba_ref, bb_ref,
                 cc0_ref, cc1_ref, cc2_ref, cc3_ref, lp_ref, lb_ref,
                 out_ref):
    WA = wa_ref[:, :]
    WB = wb_ref[:, :]
    B2 = ba_ref[:, :] + bb_ref[:, :]
    C0 = cc0_ref[:, :]
    C1 = cc1_ref[:, :]
    C2 = cc2_ref[:, :]
    C3 = cc3_ref[:, :]

    def make_block(c0_ref, c1_ref):
        def block(q, u):
            base = q * 32
            for k in range(32):
                i = base + k
                a = c0_ref[i] * WA + c1_ref[i] * WB + B2
                u1 = jnp.roll(u, -1, axis=0)
                u2 = jnp.roll(u, -2, axis=0)
                u3 = jnp.roll(u, -3, axis=0)
                z = ((a + C0 * u) + (C1 * u1 + C2 * u2)) + C3 * u3
                u = jnp.tanh(z)
            return u
        return block

    u0 = jnp.zeros((8, 1), dtype=jnp.float32)
    uh = lax.fori_loop(0, (2 * N) // 32, make_block(c0a_ref, c1a_ref), u0)
    uT = lax.fori_loop(0, (2 * N) // 32, make_block(c0b_ref, c1b_ref), uh)
    hr = jnp.maximum(uT, 0.0)
    z = jnp.sum(hr * lp_ref[:, :], axis=0, keepdims=True) + lb_ref[0:1, 0:1]
    out_ref[:, :] = 1.0 / (1.0 + jnp.exp(-z))


def _tc_rnn(c0a, c1a, c0b, c1b, wa, wb, ba, bb, cc0, cc1, cc2, cc3, lp, lb):
    smem = pl.BlockSpec(memory_space=pltpu.SMEM)
    vmem = pl.BlockSpec(memory_space=pltpu.VMEM)
    return pl.pallas_call(
        _tc_rnn_body,
        out_shape=jax.ShapeDtypeStruct((1, 1), jnp.float32),
        in_specs=[smem] * 4 + [vmem] * 10,
    )(c0a, c1a, c0b, c1b, wa, wb, ba, bb, cc0, cc1, cc2, cc3, lp, lb)


def kernel(x, edge_index, gcn_W, gcn_b, Wih, Whh, bih, bhh, lin_W, lin_b):
    f32 = jnp.float32
    ei = edge_index.astype(jnp.int32)
    src = ei[:, 0, :].reshape(T, NSH, ESH)
    dst = ei[:, 1, :].reshape(T, NSH, ESH)
    src_p = jnp.pad(src, ((0, 0), (0, 0), (0, EPAD - ESH)),
                    constant_values=0).reshape(T * NSH, NCHUNK, 128)
    dst_p = jnp.pad(dst, ((0, 0), (0, 0), (0, EPAD - ESH)),
                    constant_values=PADDST).reshape(T * NSH, NCHUNK, 128)
    x0 = x[:, :, 0].reshape(T * N)
    x1 = x[:, :, 1].reshape(T * N)
    wv = jnp.concatenate([gcn_W.reshape(4), gcn_b.reshape(2),
                          jnp.zeros((10,), f32)]).astype(f32)

    p0_out0, p0_out1 = _sc_gcn(src_p, dst_p, x0, x1, wv, 0)
    p1_out0, p1_out1 = _sc_gcn(src_p, dst_p, x0, x1, wv, 1)

    idx4 = jnp.array([0, 1, 2, 3, 0, 1, 2, 3])
    Wih4 = jnp.zeros((4, 2), f32).at[:3].set(Wih.astype(f32))
    Whh4 = jnp.zeros((4, 4), f32).at[:3, :3].set(Whh.astype(f32))
    b4a = jnp.zeros((4,), f32).at[:3].set(bih.astype(f32))
    b4b = jnp.zeros((4,), f32).at[:3].set(bhh.astype(f32))
    wa = Wih4[idx4, 0].reshape(8, 1)
    wb = Wih4[idx4, 1].reshape(8, 1)
    ba = b4a[idx4].reshape(8, 1)
    bb = b4b[idx4].reshape(8, 1)
    cc0 = Whh4[idx4, idx4].reshape(8, 1)
    cc1 = Whh4[idx4, (idx4 + 1) % 4].reshape(8, 1)
    cc2 = Whh4[idx4, (idx4 + 2) % 4].reshape(8, 1)
    cc3 = Whh4[idx4, (idx4 + 3) % 4].reshape(8, 1)
    lp = jnp.concatenate([lin_W.reshape(3).astype(f32),
                          jnp.zeros((5,), f32)]).reshape(8, 1)
    lb = lin_b.reshape(1, 1).astype(f32)

    return _tc_rnn(p0_out0, p0_out1, p1_out0, p1_out1,
                   wa, wb, ba, bb, cc0, cc1, cc2, cc3, lp, lb)


# speedup vs baseline: 1.0613x; 1.0613x over previous
import jax
import jax.numpy as jnp
from jax import lax
from jax.experimental import pallas as pl
from jax.experimental.pallas import tpu as pltpu
from jax.experimental.pallas import tpu_sc as plsc

N = 10000
T = 4
E = 160000
NSH = 16
ESH = E // NSH
NCHUNK = 79
EPAD = NCHUNK * 128
PADDST = N
ACCN = 10240
ZCH = ACCN // 16
NSL = 632
NBUF = 640
LASTSTART = N - NSL
RING = 4


def _fisr(x):
    i = lax.bitcast_convert_type(x, jnp.int32)
    i = jnp.int32(0x5F3759DF) - lax.shift_right_arithmetic(i, 1)
    y = lax.bitcast_convert_type(i, jnp.float32)
    for _ in range(3):
        y = y * (1.5 - 0.5 * x * y * y)
    return y


def _ring_scatter(n, make_copy):
    for j in range(RING):
        make_copy(j).start(add=True)

    def body(j, _):
        make_copy(j).start(add=True)
        make_copy(j - RING).wait()
        return 0
    lax.fori_loop(RING, n, body, 0)
    for j in range(n - RING, n):
        make_copy(j).wait()


def _sc_gcn_body(src_hbm, dst_hbm, x0_hbm, x1_hbm, wv_hbm,
                 out0_hbm, out1_hbm,
                 idx2d, src2d, val0, val1, g0v, g1v,
                 x0s, x1s, dinvs, gt0, gt1, a0s, a1s, outs, zbuf, wv,
                 sem0, sem1,
                 deg_sh, acc0_sh, acc1_sh, g0_sh, g1_sh, *, pair):
    c = lax.axis_index("c")
    s = lax.axis_index("s")
    t = 2 * pair + c
    start = lax.min(s * NSL, LASTSTART)

    ones16 = jnp.full((16,), 1.0, dtype=jnp.float32)
    zero16 = jnp.zeros((16,), dtype=jnp.float32)

    def zfill(m, _):
        zbuf[pl.ds(m * 16, 16)] = zero16
        return 0
    lax.fori_loop(0, ZCH // 16, zfill, 0)
    pltpu.sync_copy(zbuf, deg_sh.at[pl.ds(s * ZCH, ZCH)])
    pltpu.sync_copy(zbuf, acc0_sh.at[pl.ds(s * ZCH, ZCH)])
    pltpu.sync_copy(zbuf, acc1_sh.at[pl.ds(s * ZCH, ZCH)])

    pltpu.sync_copy(wv_hbm, wv)
    chunk = t * NSH + s
    pltpu.sync_copy(dst_hbm.at[chunk], idx2d)
    pltpu.sync_copy(src_hbm.at[chunk], src2d)
    nbase = t * N + start
    pltpu.sync_copy(x0_hbm.at[pl.ds(nbase, NSL)], x0s.at[pl.ds(0, NSL)])
    pltpu.sync_copy(x1_hbm.at[pl.ds(nbase, NSL)], x1s.at[pl.ds(0, NSL)])

    def onesfill(j, _):
        for k in range(8):
            val0[j, pl.ds(k * 16, 16)] = ones16
        return 0
    lax.fori_loop(0, NCHUNK, onesfill, 0)

    plsc.subcore_barrier()

    def deg_copy(j):
        return pltpu.make_async_copy(val0.at[j], deg_sh.at[idx2d.at[j]], sem0)
    _ring_scatter(NCHUNK, deg_copy)

    plsc.subcore_barrier()

    pltpu.sync_copy(deg_sh.at[pl.ds(start, NSL)], dinvs.at[pl.ds(0, NSL)])
    wvec = wv[pl.ds(0, 16)]
    w00 = wvec[0]
    w01 = wvec[1]
    w10 = wvec[2]
    w11 = wvec[3]

    def gbody(m, _):
        dv = dinvs[pl.ds(m * 16, 16)]
        di = _fisr(dv + 1.0)
        dinvs[pl.ds(m * 16, 16)] = di
        xa = x0s[pl.ds(m * 16, 16)]
        xb = x1s[pl.ds(m * 16, 16)]
        gt0[pl.ds(m * 16, 16)] = (xa * w00 + xb * w01) * di
        gt1[pl.ds(m * 16, 16)] = (xa * w10 + xb * w11) * di
        return 0
    lax.fori_loop(0, NBUF // 16, gbody, 0)
    pltpu.sync_copy(gt0.at[pl.ds(0, NSL)], g0_sh.at[pl.ds(start, NSL)])
    pltpu.sync_copy(gt1.at[pl.ds(0, NSL)], g1_sh.at[pl.ds(start, NSL)])

    plsc.subcore_barrier()

    pltpu.sync_copy(g0_sh, g0v)
    pltpu.sync_copy(g1_sh, g1v)

    def gathbody(j, _):
        for k in range(8):
            sv = src2d[j, pl.ds(k * 16, 16)]
            val0[j, pl.ds(k * 16, 16)] = plsc.load_gather(g0v, [sv])
            val1[j, pl.ds(k * 16, 16)] = plsc.load_gather(g1v, [sv])
        return 0
    lax.fori_loop(0, NCHUNK, gathbody, 0)

    def msg_copy0(j):
        return pltpu.make_async_copy(val0.at[j], acc0_sh.at[idx2d.at[j]], sem0)

    def msg_copy1(j):
        return pltpu.make_async_copy(val1.at[j], acc1_sh.at[idx2d.at[j]], sem1)
    _ring_scatter(NCHUNK, msg_copy0)
    _ring_scatter(NCHUNK, msg_copy1)

    plsc.subcore_barrier()

    pltpu.sync_copy(acc0_sh.at[pl.ds(start, NSL)], a0s.at[pl.ds(0, NSL)])
    pltpu.sync_copy(acc1_sh.at[pl.ds(start, NSL)], a1s.at[pl.ds(0, NSL)])
    bvec = wv[pl.ds(0, 16)]
    b0 = bvec[4]
    b1 = bvec[5]
    obase = (t - 2 * pair) * N + start

    def obody(m, _):
        di = dinvs[pl.ds(m * 16, 16)]
        outs[pl.ds(m * 16, 16)] = (
            di * (a0s[pl.ds(m * 16, 16)] + gt0[pl.ds(m * 16, 16)]) + b0)
        return 0
    lax.fori_loop(0, NBUF // 16, obody, 0)
    pltpu.sync_copy(outs.at[pl.ds(0, NSL)], out0_hbm.at[pl.ds(obase, NSL)])

    def obody1(m, _):
        di = dinvs[pl.ds(m * 16, 16)]
        outs[pl.ds(m * 16, 16)] = (
            di * (a1s[pl.ds(m * 16, 16)] + gt1[pl.ds(m * 16, 16)]) + b1)
        return 0
    lax.fori_loop(0, NBUF // 16, obody1, 0)
    pltpu.sync_copy(outs.at[pl.ds(0, NSL)], out1_hbm.at[pl.ds(obase, NSL)])


def _sc_gcn(src_p, dst_p, x0, x1, wv, pair):
    mesh = plsc.VectorSubcoreMesh(core_axis_name="c", subcore_axis_name="s",
                                  num_cores=2, num_subcores=16)
    f32 = jnp.float32

    def body(*refs):
        _sc_gcn_body(*refs, pair=pair)

    fn = pl.kernel(
        body,
        out_type=(jax.ShapeDtypeStruct((2 * N,), f32),
                  jax.ShapeDtypeStruct((2 * N,), f32)),
        mesh=mesh,
        compiler_params=pltpu.CompilerParams(needs_layout_passes=False),
        scratch_types=[
            pltpu.VMEM((NCHUNK, 128), jnp.int32),
            pltpu.VMEM((NCHUNK, 128), jnp.int32),
            pltpu.VMEM((NCHUNK, 128), f32),
            pltpu.VMEM((NCHUNK, 128), f32),
            pltpu.VMEM((N,), f32),
            pltpu.VMEM((N,), f32),
            pltpu.VMEM((NBUF,), f32),
            pltpu.VMEM((NBUF,), f32),
            pltpu.VMEM((NBUF,), f32),
            pltpu.VMEM((NBUF,), f32),
            pltpu.VMEM((NBUF,), f32),
            pltpu.VMEM((NBUF,), f32),
            pltpu.VMEM((NBUF,), f32),
            pltpu.VMEM((NBUF,), f32),
            pltpu.VMEM((ZCH,), f32),
            pltpu.VMEM((16,), f32),
            pltpu.SemaphoreType.DMA,
            pltpu.SemaphoreType.DMA,
            pltpu.VMEM_SHARED((ACCN,), f32),
            pltpu.VMEM_SHARED((ACCN,), f32),
            pltpu.VMEM_SHARED((ACCN,), f32),
            pltpu.VMEM_SHARED((N,), f32),
            pltpu.VMEM_SHARED((N,), f32),
        ],
    )
    return fn(src_p, dst_p, x0, x1, wv)


def _tc_rnn_body(c0_ref, c1_ref, hin_ref, wpack_ref, hout_ref, out_ref):
    WA = wpack_ref[0:8, :]
    WB = wpack_ref[8:16, :]
    B2 = wpack_ref[16:24, :]
    C0 = wpack_ref[24:32, :]
    C1 = wpack_ref[32:40, :]
    C2 = wpack_ref[40:48, :]
    C3 = wpack_ref[48:56, :]
    LP = wpack_ref[56:64, :]
    lb = wpack_ref[64:65, :]

    def block(q, u):
        base = q * 32
        for k in range(32):
            i = base + k
            a = c0_ref[i] * WA + c1_ref[i] * WB + B2
            u1 = jnp.roll(u, -1, axis=0)
            u2 = jnp.roll(u, -2, axis=0)
            u3 = jnp.roll(u, -3, axis=0)
            z = ((a + C0 * u) + (C1 * u1 + C2 * u2)) + C3 * u3
            u = jnp.tanh(z)
        return u

    u0 = hin_ref[:, :]
    uT = lax.fori_loop(0, (2 * N) // 32, block, u0)
    hout_ref[:, :] = uT
    hr = jnp.maximum(uT, 0.0)
    z = jnp.sum(hr * LP, axis=0, keepdims=True) + lb
    out_ref[:, :] = 1.0 / (1.0 + jnp.exp(-z))


def _tc_rnn(c0, c1, hin, wpack):
    smem = pl.BlockSpec(memory_space=pltpu.SMEM)
    vmem = pl.BlockSpec(memory_space=pltpu.VMEM)
    return pl.pallas_call(
        _tc_rnn_body,
        out_shape=(jax.ShapeDtypeStruct((8, 1), jnp.float32),
                   jax.ShapeDtypeStruct((1, 1), jnp.float32)),
        in_specs=[smem, smem, vmem, vmem],
    )(c0, c1, hin, wpack)


def kernel(x, edge_index, gcn_W, gcn_b, Wih, Whh, bih, bhh, lin_W, lin_b):
    f32 = jnp.float32
    ei = edge_index.astype(jnp.int32)
    src = ei[:, 0, :].reshape(T, NSH, ESH)
    dst = ei[:, 1, :].reshape(T, NSH, ESH)
    src_p = jnp.pad(src, ((0, 0), (0, 0), (0, EPAD - ESH)),
                    constant_values=0).reshape(T * NSH, NCHUNK, 128)
    dst_p = jnp.pad(dst, ((0, 0), (0, 0), (0, EPAD - ESH)),
                    constant_values=PADDST).reshape(T * NSH, NCHUNK, 128)
    x0 = x[:, :, 0].reshape(T * N)
    x1 = x[:, :, 1].reshape(T * N)
    wv = jnp.concatenate([gcn_W.reshape(4), gcn_b.reshape(2),
                          jnp.zeros((10,), f32)]).astype(f32)

    p0_out0, p0_out1 = _sc_gcn(src_p, dst_p, x0, x1, wv, 0)
    p1_out0, p1_out1 = _sc_gcn(src_p, dst_p, x0, x1, wv, 1)

    idx4 = jnp.array([0, 1, 2, 3, 0, 1, 2, 3])
    Wih4 = jnp.zeros((4, 2), f32).at[:3].set(Wih.astype(f32))
    Whh4 = jnp.zeros((4, 4), f32).at[:3, :3].set(Whh.astype(f32))
    b4 = jnp.zeros((4,), f32).at[:3].set((bih + bhh).astype(f32))
    wpack = jnp.concatenate([
        Wih4[idx4, 0], Wih4[idx4, 1], b4[idx4],
        Whh4[idx4, idx4], Whh4[idx4, (idx4 + 1) % 4],
        Whh4[idx4, (idx4 + 2) % 4], Whh4[idx4, (idx4 + 3) % 4],
        lin_W.reshape(3).astype(f32), jnp.zeros((5,), f32),
        lin_b.reshape(1).astype(f32), jnp.zeros((7,), f32),
    ]).reshape(72, 1)

    h0 = jnp.zeros((8, 1), f32)
    h1, _ = _tc_rnn(p0_out0, p0_out1, h0, wpack)
    _, out = _tc_rnn(p1_out0, p1_out1, h1, wpack)
    return out
